# Initial kernel scaffold; baseline (speedup 1.0000x reference)
import jax
import jax.numpy as jnp
from jax.experimental import pallas as pl


def kernel(voxel_feat, batch_idx, global_feat, global_weight, w_glob, b_glob, w1, b1, w2, b2):
    raise NotImplementedError("write your pallas kernel here")


# speedup vs baseline: 1.7168x; 1.7168x over previous
import functools

import jax
import jax.numpy as jnp
from jax.experimental import pallas as pl
from jax.experimental.pallas import tpu as pltpu


_VMEM_LIMIT_BYTES = 48 * 1024 * 1024


def _cdiv(a, b):
    return (a + b - 1) // b


def _divisor_tile(n, cap):
    t = min(int(cap), max(int(n), 8)) // 8 * 8
    while t >= 8:
        if n % t == 0:
            return t
        t -= 8
    return 0


def _pool_kernel(vf_ref, bidx_ref, sum_ref, cnt_ref, *, n_rows, n_half,
                 row_tile, n_batches, ragged):
    core = pl.program_id(0)
    step = pl.program_id(1)

    @pl.when(step == 0)
    def _():
        sum_ref[...] = jnp.zeros_like(sum_ref)
        cnt_ref[...] = jnp.zeros_like(cnt_ref)

    tile = core * n_half + step
    vf = vf_ref[...]
    if ragged:
        row = tile * row_tile + jax.lax.broadcasted_iota(
            jnp.int32, (row_tile, 1), 0)
        vf = jnp.where(row < n_rows, vf, 0.0)

    bidx = bidx_ref[...].reshape(1, row_tile)
    col = tile * row_tile + jax.lax.broadcasted_iota(
        jnp.int32, (n_batches, row_tile), 1)
    hit = (bidx == jax.lax.broadcasted_iota(
        jnp.int32, (n_batches, row_tile), 0)) & (col < n_rows)
    onehot = jnp.where(hit, 1.0, 0.0)

    sum_ref[...] += jax.lax.dot_general(
        onehot, vf, (((1,), (0,)), ((), ())),
        preferred_element_type=jnp.float32)[None]
    cnt_ref[...] += jnp.sum(onehot, axis=1, keepdims=True)[None]


def _scale_kernel(vf_ref, code_ref, smat_ref, out_ref, *, fold, n_batches):
    codes = code_ref[...]
    t = codes.shape[0]
    iota = jax.lax.broadcasted_iota(
        jnp.int32, (t, fold * n_batches), 1)
    hit = iota == codes[:, 0:1]
    for s in range(1, fold):
        hit = hit | (iota == codes[:, s:s + 1] + s * n_batches)
    onehot = jnp.where(hit, 1.0, 0.0)
    scale = jax.lax.dot_general(
        onehot, smat_ref[...], (((1,), (0,)), ((), ())),
        preferred_element_type=jnp.float32)
    out_ref[...] = vf_ref[...] * scale


def kernel(voxel_feat, batch_idx, global_feat, global_weight,
           w_glob, b_glob, w1, b1, w2, b2):
    n_voxels, channels = voxel_feat.shape
    n_batches = global_feat.shape[0]
    f32 = jnp.float32
    hi = jax.lax.Precision.HIGHEST
    batch_idx = batch_idx.astype(jnp.int32)

    t1 = _divisor_tile(n_voxels, 8192)
    ragged1 = t1 == 0
    if ragged1:
        t1 = min(8192, _cdiv(n_voxels, 8) * 8)
    n_tiles = _cdiv(n_voxels, t1)
    n_splits = 2 if n_tiles > 1 else 1
    n_half = _cdiv(n_tiles, n_splits)

    pad = n_tiles * t1 - n_voxels
    bidx_lane = batch_idx if pad == 0 else jnp.pad(
        batch_idx, (0, pad), constant_values=n_batches)
    bidx_lane = bidx_lane.reshape(n_tiles, 1, t1)

    def _tile_idx(c, s):
        return jnp.minimum(c * n_half + s, n_tiles - 1)

    sums, cnts = pl.pallas_call(
        functools.partial(_pool_kernel, n_rows=n_voxels, n_half=n_half,
                          row_tile=t1, n_batches=n_batches, ragged=ragged1),
        grid=(n_splits, n_half),
        in_specs=[
            pl.BlockSpec((t1, channels), lambda c, s: (_tile_idx(c, s), 0)),
            pl.BlockSpec((1, 1, t1), lambda c, s: (_tile_idx(c, s), 0, 0)),
        ],
        out_specs=[
            pl.BlockSpec((1, n_batches, channels), lambda c, s: (c, 0, 0)),
            pl.BlockSpec((1, n_batches, 1), lambda c, s: (c, 0, 0)),
        ],
        out_shape=[
            jax.ShapeDtypeStruct((n_splits, n_batches, channels), f32),
            jax.ShapeDtypeStruct((n_splits, n_batches, 1), f32),
        ],
        compiler_params=pltpu.CompilerParams(
            dimension_semantics=("parallel", "arbitrary"),
            vmem_limit_bytes=_VMEM_LIMIT_BYTES),
    )(voxel_feat, bidx_lane)

    pooled = sums.sum(axis=0) / jnp.maximum(cnts.sum(axis=0), 1.0)
    gt = (jnp.dot(global_feat.astype(f32), w_glob.T.astype(f32), precision=hi)
          + b_glob.astype(f32))
    gt = jnp.asarray(global_weight, f32) * gt
    combined = jnp.concatenate([pooled, gt], axis=1)
    hidden = jax.nn.gelu(
        jnp.dot(combined, w1.T.astype(f32), precision=hi)
        + b1.astype(f32), approximate=False)
    scaling = jax.nn.sigmoid(
        jnp.dot(hidden, w2.T.astype(f32), precision=hi)
        + b2.astype(f32))

    if channels < 128 and 128 % channels == 0 and \
            n_voxels % (128 // channels) == 0:
        fold = 128 // channels
    else:
        fold = 1
    n_rows2 = n_voxels // fold
    width = fold * channels
    vf2 = voxel_feat.reshape(n_rows2, width)
    codes = batch_idx.reshape(n_rows2, fold)
    smat = jnp.kron(jnp.eye(fold, dtype=f32), scaling)

    t2 = _divisor_tile(n_rows2, 8192)
    if t2 == 0:
        t2 = min(8192, _cdiv(n_rows2, 8) * 8)
    n_tiles2 = _cdiv(n_rows2, t2)
    k_dim = fold * n_batches

    out = pl.pallas_call(
        functools.partial(_scale_kernel, fold=fold, n_batches=n_batches),
        grid=(n_tiles2,),
        in_specs=[
            pl.BlockSpec((t2, width), lambda i: (i, 0)),
            pl.BlockSpec((t2, fold), lambda i: (i, 0)),
            pl.BlockSpec((k_dim, width), lambda i: (0, 0)),
        ],
        out_specs=pl.BlockSpec((t2, width), lambda i: (i, 0)),
        out_shape=jax.ShapeDtypeStruct((n_rows2, width), voxel_feat.dtype),
        compiler_params=pltpu.CompilerParams(
            dimension_semantics=("parallel",),
            vmem_limit_bytes=_VMEM_LIMIT_BYTES),
    )(vf2, codes, smat)

    return out.reshape(n_voxels, channels), scaling


# speedup vs baseline: 2.9588x; 1.7235x over previous
import functools

import jax
import jax.numpy as jnp
from jax.experimental import pallas as pl
from jax.experimental.pallas import tpu as pltpu


_VMEM_LIMIT_BYTES = 48 * 1024 * 1024


def _cdiv(a, b):
    return (a + b - 1) // b


def _divisor_tile(n, cap):
    t = min(int(cap), max(int(n), 8)) // 8 * 8
    while t >= 8:
        if n % t == 0:
            return t
        t -= 8
    return 0


def _pool_kernel(vf_ref, bidx_ref, sum_ref, cnt_ref, *, n_rows, n_half,
                 row_tile, n_batches, ragged):
    core = pl.program_id(0)
    step = pl.program_id(1)

    @pl.when(step == 0)
    def _():
        sum_ref[...] = jnp.zeros_like(sum_ref)
        cnt_ref[...] = jnp.zeros_like(cnt_ref)

    tile = core * n_half + step
    vf = vf_ref[...]
    if ragged:
        row = tile * row_tile + jax.lax.broadcasted_iota(
            jnp.int32, (row_tile, 1), 0)
        vf = jnp.where(row < n_rows, vf, 0.0)

    bidx = bidx_ref[...].reshape(1, row_tile)
    col = tile * row_tile + jax.lax.broadcasted_iota(
        jnp.int32, (n_batches, row_tile), 1)
    hit = (bidx == jax.lax.broadcasted_iota(
        jnp.int32, (n_batches, row_tile), 0)) & (col < n_rows)
    onehot = jnp.where(hit, 1.0, 0.0)

    sum_ref[...] += jax.lax.dot_general(
        onehot, vf, (((1,), (0,)), ((), ())),
        preferred_element_type=jnp.float32)[None]
    cnt_ref[...] += jnp.sum(onehot, axis=1, keepdims=True)[None]


def _scale_kernel(vf_ref, bidx_ref, scal_ref, out_ref, *, row_tile, n_batches):
    bidx = bidx_ref[...].reshape(1, row_tile)
    onehot = jnp.where(
        bidx == jax.lax.broadcasted_iota(
            jnp.int32, (n_batches, row_tile), 0), 1.0, 0.0)
    scale = jax.lax.dot_general(
        onehot, scal_ref[...], (((0,), (0,)), ((), ())),
        preferred_element_type=jnp.float32)
    out_ref[...] = vf_ref[...] * scale


def kernel(voxel_feat, batch_idx, global_feat, global_weight,
           w_glob, b_glob, w1, b1, w2, b2):
    n_voxels, channels = voxel_feat.shape
    n_batches = global_feat.shape[0]
    f32 = jnp.float32
    hi = jax.lax.Precision.HIGHEST
    batch_idx = batch_idx.astype(jnp.int32)

    t1 = _divisor_tile(n_voxels, 8192)
    ragged = t1 == 0
    if ragged:
        t1 = min(8192, _cdiv(n_voxels, 8) * 8)
    n_tiles = _cdiv(n_voxels, t1)
    n_splits = 2 if n_tiles > 1 else 1
    n_half = _cdiv(n_tiles, n_splits)

    pad = n_tiles * t1 - n_voxels
    bidx_lane = batch_idx if pad == 0 else jnp.pad(
        batch_idx, (0, pad), constant_values=n_batches)
    bidx_lane = bidx_lane.reshape(n_tiles, 1, t1)

    def _tile_idx(c, s):
        return jnp.minimum(c * n_half + s, n_tiles - 1)

    sums, cnts = pl.pallas_call(
        functools.partial(_pool_kernel, n_rows=n_voxels, n_half=n_half,
                          row_tile=t1, n_batches=n_batches, ragged=ragged),
        grid=(n_splits, n_half),
        in_specs=[
            pl.BlockSpec((t1, channels), lambda c, s: (_tile_idx(c, s), 0)),
            pl.BlockSpec((1, 1, t1), lambda c, s: (_tile_idx(c, s), 0, 0)),
        ],
        out_specs=[
            pl.BlockSpec((1, n_batches, channels), lambda c, s: (c, 0, 0)),
            pl.BlockSpec((1, n_batches, 1), lambda c, s: (c, 0, 0)),
        ],
        out_shape=[
            jax.ShapeDtypeStruct((n_splits, n_batches, channels), f32),
            jax.ShapeDtypeStruct((n_splits, n_batches, 1), f32),
        ],
        compiler_params=pltpu.CompilerParams(
            dimension_semantics=("parallel", "arbitrary"),
            vmem_limit_bytes=_VMEM_LIMIT_BYTES),
    )(voxel_feat, bidx_lane)

    pooled = sums.sum(axis=0) / jnp.maximum(cnts.sum(axis=0), 1.0)
    gt = (jnp.dot(global_feat.astype(f32), w_glob.T.astype(f32), precision=hi)
          + b_glob.astype(f32))
    gt = jnp.asarray(global_weight, f32) * gt
    combined = jnp.concatenate([pooled, gt], axis=1)
    hidden = jax.nn.gelu(
        jnp.dot(combined, w1.T.astype(f32), precision=hi)
        + b1.astype(f32), approximate=False)
    scaling = jax.nn.sigmoid(
        jnp.dot(hidden, w2.T.astype(f32), precision=hi)
        + b2.astype(f32))

    out = pl.pallas_call(
        functools.partial(_scale_kernel, row_tile=t1, n_batches=n_batches),
        grid=(n_tiles,),
        in_specs=[
            pl.BlockSpec((t1, channels), lambda i: (i, 0)),
            pl.BlockSpec((1, 1, t1), lambda i: (i, 0, 0)),
            pl.BlockSpec((n_batches, channels), lambda i: (0, 0)),
        ],
        out_specs=pl.BlockSpec((t1, channels), lambda i: (i, 0)),
        out_shape=jax.ShapeDtypeStruct((n_voxels, channels),
                                       voxel_feat.dtype),
        compiler_params=pltpu.CompilerParams(
            dimension_semantics=("parallel",),
            vmem_limit_bytes=_VMEM_LIMIT_BYTES),
    )(voxel_feat, bidx_lane, scaling)

    return out, scaling


# speedup vs baseline: 3.0249x; 1.0223x over previous
import functools

import jax
import jax.numpy as jnp
from jax.experimental import pallas as pl
from jax.experimental.pallas import tpu as pltpu


_VMEM_LIMIT_BYTES = 48 * 1024 * 1024


def _cdiv(a, b):
    return (a + b - 1) // b


def _divisor_tile(n, cap):
    t = min(int(cap), max(int(n), 8)) // 8 * 8
    while t >= 8:
        if n % t == 0:
            return t
        t -= 8
    return 0


def _pool_kernel(vf_ref, bidx_ref, sum_ref, cnt_ref, *, n_rows, row_tile,
                 n_batches, ragged):
    tile = pl.program_id(0)
    vf = vf_ref[...]
    if ragged:
        row = tile * row_tile + jax.lax.broadcasted_iota(
            jnp.int32, (row_tile, 1), 0)
        vf = jnp.where(row < n_rows, vf, 0.0)

    bidx = bidx_ref[...].reshape(1, row_tile)
    onehot = jnp.where(
        bidx == jax.lax.broadcasted_iota(
            jnp.int32, (n_batches, row_tile), 0), 1.0, 0.0)

    sum_ref[...] = jax.lax.dot_general(
        onehot, vf, (((1,), (0,)), ((), ())),
        preferred_element_type=jnp.float32)[None]
    cnt_ref[...] = jnp.sum(onehot, axis=1, keepdims=True)[None]


def _scale_kernel(vf_ref, bidx_ref, scal_ref, out_ref, *, row_tile, n_batches):
    bidx = bidx_ref[...].reshape(1, row_tile)
    onehot = jnp.where(
        bidx == jax.lax.broadcasted_iota(
            jnp.int32, (n_batches, row_tile), 0), 1.0, 0.0)
    scale = jax.lax.dot_general(
        onehot, scal_ref[...], (((0,), (0,)), ((), ())),
        preferred_element_type=jnp.float32)
    out_ref[...] = vf_ref[...] * scale


def kernel(voxel_feat, batch_idx, global_feat, global_weight,
           w_glob, b_glob, w1, b1, w2, b2):
    n_voxels, channels = voxel_feat.shape
    n_batches = global_feat.shape[0]
    f32 = jnp.float32
    hi = jax.lax.Precision.HIGHEST
    batch_idx = batch_idx.astype(jnp.int32)

    t1 = _divisor_tile(n_voxels, 20000)
    ragged = t1 == 0
    if ragged:
        t1 = min(20000 // 8 * 8, _cdiv(n_voxels, 8) * 8)
    n_tiles = _cdiv(n_voxels, t1)

    pad = n_tiles * t1 - n_voxels
    bidx_lane = batch_idx if pad == 0 else jnp.pad(
        batch_idx, (0, pad), constant_values=n_batches)
    bidx_lane = bidx_lane.reshape(n_tiles, 1, t1)

    sums, cnts = pl.pallas_call(
        functools.partial(_pool_kernel, n_rows=n_voxels,
                          row_tile=t1, n_batches=n_batches, ragged=ragged),
        grid=(n_tiles,),
        in_specs=[
            pl.BlockSpec((t1, channels), lambda i: (i, 0)),
            pl.BlockSpec((1, 1, t1), lambda i: (i, 0, 0)),
        ],
        out_specs=[
            pl.BlockSpec((1, n_batches, channels), lambda i: (i, 0, 0)),
            pl.BlockSpec((1, n_batches, 1), lambda i: (i, 0, 0)),
        ],
        out_shape=[
            jax.ShapeDtypeStruct((n_tiles, n_batches, channels), f32),
            jax.ShapeDtypeStruct((n_tiles, n_batches, 1), f32),
        ],
        compiler_params=pltpu.CompilerParams(
            dimension_semantics=("parallel",),
            vmem_limit_bytes=_VMEM_LIMIT_BYTES),
    )(voxel_feat, bidx_lane)

    pooled = sums.sum(axis=0) / jnp.maximum(cnts.sum(axis=0), 1.0)
    gt = (jnp.dot(global_feat.astype(f32), w_glob.T.astype(f32), precision=hi)
          + b_glob.astype(f32))
    gt = jnp.asarray(global_weight, f32) * gt
    combined = jnp.concatenate([pooled, gt], axis=1)
    hidden = jax.nn.gelu(
        jnp.dot(combined, w1.T.astype(f32), precision=hi)
        + b1.astype(f32), approximate=False)
    scaling = jax.nn.sigmoid(
        jnp.dot(hidden, w2.T.astype(f32), precision=hi)
        + b2.astype(f32))

    out = pl.pallas_call(
        functools.partial(_scale_kernel, row_tile=t1, n_batches=n_batches),
        grid=(n_tiles,),
        in_specs=[
            pl.BlockSpec((t1, channels), lambda i: (i, 0)),
            pl.BlockSpec((1, 1, t1), lambda i: (i, 0, 0)),
            pl.BlockSpec((n_batches, channels), lambda i: (0, 0)),
        ],
        out_specs=pl.BlockSpec((t1, channels), lambda i: (i, 0)),
        out_shape=jax.ShapeDtypeStruct((n_voxels, channels),
                                       voxel_feat.dtype),
        compiler_params=pltpu.CompilerParams(
            dimension_semantics=("parallel",),
            vmem_limit_bytes=_VMEM_LIMIT_BYTES),
    )(voxel_feat, bidx_lane, scaling)

    return out, scaling
